# baseline (speedup 1.0000x reference)
import dataclasses

import jax
import jax.numpy as jnp
from jax import lax
from jax.experimental import pallas as pl
from jax.experimental.pallas import tpu as pltpu
from jax.experimental.pallas import tpu_sc as plsc

N = 10000
NP = 10240
E = 320000
D = 128

NC = 2
NS = 16
L = 16
NW = NC * NS
EB = 64
NB = 160
EPT = NB * EB
EPAD = NW * EPT
RPT = NP // NS
GQ = 4
IQ = 8

_mesh = plsc.VectorSubcoreMesh(core_axis_name="c", subcore_axis_name="s")


def _make_sc_agg():
    scratch = (
        [pltpu.VMEM((EB,), jnp.int32) for _ in range(IQ)]
        + [pltpu.VMEM((EB,), jnp.int32) for _ in range(IQ)]
        + [pltpu.VMEM((EB, D), jnp.float32) for _ in range(GQ)]
        + [pltpu.VMEM_SHARED((NP, D), jnp.float32)]
        + [pltpu.SemaphoreType.DMA for _ in range(GQ + IQ)]
    )

    def body(h_hbm, row_hbm, col_hbm, out_hbm, *rest):
        rcur = rest[:IQ]
        ccur = rest[IQ:2 * IQ]
        bufs = rest[2 * IQ:2 * IQ + GQ]
        acc = rest[2 * IQ + GQ]
        gsem = rest[2 * IQ + GQ + 1:2 * IQ + GQ + 1 + GQ]
        isem = rest[2 * IQ + GQ + 1 + GQ:]

        cid = lax.axis_index("c")
        sid = lax.axis_index("s")
        wid = sid * NC + cid

        zval = jnp.zeros((L,), jnp.float32)

        @pl.loop(0, EB)
        def _(r):
            @pl.loop(0, D, step=L)
            def _(c0):
                bufs[0][r, pl.ds(c0, L)] = zval

        @pl.loop(0, RPT // EB)
        def _(z):
            base = pl.multiple_of(sid * RPT + z * EB, EB)
            pltpu.sync_copy(bufs[0], acc.at[pl.ds(base, EB)])

        plsc.subcore_barrier()

        def idx_start(b, sl):
            base = pl.multiple_of(wid * EPT + b * EB, EB)
            pltpu.async_copy(row_hbm.at[pl.ds(base, EB)], rcur[sl], isem[sl])
            pltpu.async_copy(col_hbm.at[pl.ds(base, EB)], ccur[sl], isem[sl])

        def idx_wait(sl):
            pltpu.make_async_copy(
                row_hbm.at[pl.ds(0, EB)], rcur[sl], isem[sl]).wait()
            pltpu.make_async_copy(
                col_hbm.at[pl.ds(0, EB)], ccur[sl], isem[sl]).wait()

        def gather_start(bs, isl):
            pltpu.async_copy(h_hbm.at[rcur[isl]], bufs[bs], gsem[bs])

        def gather_wait(bs, isl):
            pltpu.make_async_copy(h_hbm.at[rcur[isl]], bufs[bs], gsem[bs]).wait()

        def scatter(bs, isl):
            pltpu.sync_copy(bufs[bs], acc.at[ccur[isl]], add=True)

        for sl in range(IQ):
            idx_start(sl, sl)
        for j in range(GQ):
            idx_wait(j)
            gather_start(j, j)

        @pl.loop(0, NB // IQ)
        def _(m):
            for j in range(IQ):
                b = IQ * m + j
                bs = j % GQ
                gather_wait(bs, j)
                scatter(bs, j)

                @pl.when(b + GQ < NB)
                def _():
                    idx_wait((j + GQ) % IQ)
                    gather_start(bs, (j + GQ) % IQ)

                @pl.when(b + IQ < NB)
                def _():
                    idx_start(b + IQ, j)

        plsc.subcore_barrier()

        base = pl.multiple_of(sid * RPT, RPT)
        pltpu.sync_copy(acc.at[pl.ds(base, RPT)], out_hbm.at[cid, pl.ds(base, RPT)])

    out_type = jax.ShapeDtypeStruct((NC, NP, D), jnp.float32)
    return pl.kernel(body, out_type=out_type, mesh=_mesh, scratch_types=scratch)


_sc_agg = _make_sc_agg()


def _sc_hist_body(col_hbm, out_hbm, cidx, hist):
    cid = lax.axis_index("c")
    sid = lax.axis_index("s")
    wid = sid * NC + cid

    zval = jnp.zeros((L,), jnp.float32)

    @pl.loop(0, NP, step=L)
    def _(i):
        hist[pl.ds(i, L)] = zval

    base = pl.multiple_of(wid * EPT, EPT)
    pltpu.sync_copy(col_hbm.at[pl.ds(base, EPT)], cidx)
    ones = jnp.ones((L,), jnp.float32)

    @pl.loop(0, EPT, step=L)
    def _(e):
        idx = cidx[pl.ds(e, L)]
        plsc.addupdate_scatter(hist, [idx], ones)

    obase = pl.multiple_of(wid * NP, NP)
    pltpu.sync_copy(hist, out_hbm.at[pl.ds(obase, NP)])


_sc_cp = pltpu.CompilerParams()
if "needs_layout_passes" in pltpu.CompilerParams.__dataclass_fields__:
    _sc_cp = dataclasses.replace(_sc_cp, needs_layout_passes=False)

_sc_hist = pl.kernel(
    _sc_hist_body,
    out_type=jax.ShapeDtypeStruct((NW * NP,), jnp.float32),
    mesh=_mesh,
    scratch_types=[
        pltpu.VMEM((EPT,), jnp.int32),
        pltpu.VMEM((NP,), jnp.float32),
    ],
    compiler_params=_sc_cp,
)


_RB = 1024
_G = NP // _RB


def _tc_scale_body(hist, x, w, h1p, dis):
    ones32 = jnp.ones((NW, 1), jnp.float32)
    cnt = lax.dot_general(hist[...], ones32,
                          dimension_numbers=(((0,), (0,)), ((), ())),
                          preferred_element_type=jnp.float32)
    d = lax.rsqrt(cnt + 1.0)
    h = jnp.dot(x[...], w[...], preferred_element_type=jnp.float32)
    h1p[...] = h * d
    dis[...] = d


_tc_scale = pl.pallas_call(
    _tc_scale_body,
    grid=(_G,),
    in_specs=[
        pl.BlockSpec((NW, _RB), lambda i: (0, i)),
        pl.BlockSpec((_RB, D), lambda i: (i, 0)),
        pl.BlockSpec((D, D), lambda i: (0, 0)),
    ],
    out_specs=[
        pl.BlockSpec((_RB, D), lambda i: (i, 0)),
        pl.BlockSpec((_RB, 1), lambda i: (i, 0)),
    ],
    out_shape=[
        jax.ShapeDtypeStruct((N, D), jnp.float32),
        jax.ShapeDtypeStruct((N, 1), jnp.float32),
    ],
)


def _tc_mid_body(aggp, h1p, dis, b1, w2, h2p):
    s = aggp[0] + aggp[1] + h1p[...]
    out1 = s * dis[...] + b1[...]
    h2 = jnp.dot(out1, w2[...], preferred_element_type=jnp.float32)
    h2p[...] = h2 * dis[...]


_tc_mid = pl.pallas_call(
    _tc_mid_body,
    grid=(_G,),
    in_specs=[
        pl.BlockSpec((NC, _RB, D), lambda i: (0, i, 0)),
        pl.BlockSpec((_RB, D), lambda i: (i, 0)),
        pl.BlockSpec((_RB, 1), lambda i: (i, 0)),
        pl.BlockSpec((1, D), lambda i: (0, 0)),
        pl.BlockSpec((D, D), lambda i: (0, 0)),
    ],
    out_specs=pl.BlockSpec((_RB, D), lambda i: (i, 0)),
    out_shape=jax.ShapeDtypeStruct((N, D), jnp.float32),
)


def _tc_final_body(aggp, h2p, dis, b2, out):
    s = aggp[0] + aggp[1] + h2p[...]
    z = s * dis[...] + b2[...]
    m = jnp.max(z, axis=-1, keepdims=True)
    lse = jnp.log(jnp.sum(jnp.exp(z - m), axis=-1, keepdims=True)) + m
    out[...] = z - lse


_tc_final = pl.pallas_call(
    _tc_final_body,
    grid=(_G,),
    in_specs=[
        pl.BlockSpec((NC, _RB, D), lambda i: (0, i, 0)),
        pl.BlockSpec((_RB, D), lambda i: (i, 0)),
        pl.BlockSpec((_RB, 1), lambda i: (i, 0)),
        pl.BlockSpec((1, D), lambda i: (0, 0)),
    ],
    out_specs=pl.BlockSpec((_RB, D), lambda i: (i, 0)),
    out_shape=jax.ShapeDtypeStruct((N, D), jnp.float32),
)


def kernel(x, edge_index, W1, b1, W2, b2):
    ei = edge_index.astype(jnp.int32)
    row = ei[0]
    col = ei[1]
    npad = EPAD - E
    spread = jnp.arange(npad, dtype=jnp.int32)
    rowp = jnp.concatenate([row, spread % N])
    colp = jnp.concatenate([col, N + spread % (NP - N)])
    hist = _sc_hist(colp).reshape(NW, NP)
    h1p, dis = _tc_scale(hist, x, W1)
    agg1 = _sc_agg(h1p, rowp, colp)
    h2p = _tc_mid(agg1, h1p, dis, b1.reshape(1, D), W2)
    agg2 = _sc_agg(h2p, rowp, colp)
    return _tc_final(agg2, h2p, dis, b2.reshape(1, D))


# speedup vs baseline: 1.0007x; 1.0007x over previous
import dataclasses

import jax
import jax.numpy as jnp
from jax import lax
from jax.experimental import pallas as pl
from jax.experimental.pallas import tpu as pltpu
from jax.experimental.pallas import tpu_sc as plsc

N = 10000
NP = 10240
E = 320000
D = 128

NC = 2
NS = 16
L = 16
NW = NC * NS
EB = 80
NB = 128
EPT = NB * EB
EPAD = NW * EPT
RPT = NP // NS
GQ = 4
IQ = 8

_mesh = plsc.VectorSubcoreMesh(core_axis_name="c", subcore_axis_name="s")


def _make_sc_agg():
    scratch = (
        [pltpu.VMEM((EB,), jnp.int32) for _ in range(IQ)]
        + [pltpu.VMEM((EB,), jnp.int32) for _ in range(IQ)]
        + [pltpu.VMEM((EB, D), jnp.float32) for _ in range(GQ)]
        + [pltpu.VMEM_SHARED((NP, D), jnp.float32)]
        + [pltpu.SemaphoreType.DMA for _ in range(GQ + IQ)]
    )

    def body(h_hbm, row_hbm, col_hbm, out_hbm, *rest):
        rcur = rest[:IQ]
        ccur = rest[IQ:2 * IQ]
        bufs = rest[2 * IQ:2 * IQ + GQ]
        acc = rest[2 * IQ + GQ]
        gsem = rest[2 * IQ + GQ + 1:2 * IQ + GQ + 1 + GQ]
        isem = rest[2 * IQ + GQ + 1 + GQ:]

        cid = lax.axis_index("c")
        sid = lax.axis_index("s")
        wid = sid * NC + cid

        zval = jnp.zeros((L,), jnp.float32)

        @pl.loop(0, EB)
        def _(r):
            @pl.loop(0, D, step=L)
            def _(c0):
                bufs[0][r, pl.ds(c0, L)] = zval

        @pl.loop(0, RPT // EB)
        def _(z):
            base = pl.multiple_of(sid * RPT + z * EB, EB)
            pltpu.sync_copy(bufs[0], acc.at[pl.ds(base, EB)])

        plsc.subcore_barrier()

        def idx_start(b, sl):
            base = pl.multiple_of(wid * EPT + b * EB, EB)
            pltpu.async_copy(row_hbm.at[pl.ds(base, EB)], rcur[sl], isem[sl])
            pltpu.async_copy(col_hbm.at[pl.ds(base, EB)], ccur[sl], isem[sl])

        def idx_wait(sl):
            pltpu.make_async_copy(
                row_hbm.at[pl.ds(0, EB)], rcur[sl], isem[sl]).wait()
            pltpu.make_async_copy(
                col_hbm.at[pl.ds(0, EB)], ccur[sl], isem[sl]).wait()

        def gather_start(bs, isl):
            pltpu.async_copy(h_hbm.at[rcur[isl]], bufs[bs], gsem[bs])

        def gather_wait(bs, isl):
            pltpu.make_async_copy(h_hbm.at[rcur[isl]], bufs[bs], gsem[bs]).wait()

        def scatter(bs, isl):
            pltpu.sync_copy(bufs[bs], acc.at[ccur[isl]], add=True)

        for sl in range(IQ):
            idx_start(sl, sl)
        for j in range(GQ):
            idx_wait(j)
            gather_start(j, j)

        @pl.loop(0, NB // IQ)
        def _(m):
            for j in range(IQ):
                b = IQ * m + j
                bs = j % GQ
                gather_wait(bs, j)
                scatter(bs, j)

                @pl.when(b + GQ < NB)
                def _():
                    idx_wait((j + GQ) % IQ)
                    gather_start(bs, (j + GQ) % IQ)

                @pl.when(b + IQ < NB)
                def _():
                    idx_start(b + IQ, j)

        plsc.subcore_barrier()

        base = pl.multiple_of(sid * RPT, RPT)
        pltpu.sync_copy(acc.at[pl.ds(base, RPT)], out_hbm.at[cid, pl.ds(base, RPT)])

    out_type = jax.ShapeDtypeStruct((NC, NP, D), jnp.float32)
    return pl.kernel(body, out_type=out_type, mesh=_mesh, scratch_types=scratch)


_sc_agg = _make_sc_agg()


def _sc_hist_body(col_hbm, out_hbm, cidx, hist):
    cid = lax.axis_index("c")
    sid = lax.axis_index("s")
    wid = sid * NC + cid

    zval = jnp.zeros((L,), jnp.float32)

    @pl.loop(0, NP, step=L)
    def _(i):
        hist[pl.ds(i, L)] = zval

    base = pl.multiple_of(wid * EPT, EPT)
    pltpu.sync_copy(col_hbm.at[pl.ds(base, EPT)], cidx)
    ones = jnp.ones((L,), jnp.float32)

    @pl.loop(0, EPT, step=L)
    def _(e):
        idx = cidx[pl.ds(e, L)]
        plsc.addupdate_scatter(hist, [idx], ones)

    obase = pl.multiple_of(wid * NP, NP)
    pltpu.sync_copy(hist, out_hbm.at[pl.ds(obase, NP)])


_sc_cp = pltpu.CompilerParams()
if "needs_layout_passes" in pltpu.CompilerParams.__dataclass_fields__:
    _sc_cp = dataclasses.replace(_sc_cp, needs_layout_passes=False)

_sc_hist = pl.kernel(
    _sc_hist_body,
    out_type=jax.ShapeDtypeStruct((NW * NP,), jnp.float32),
    mesh=_mesh,
    scratch_types=[
        pltpu.VMEM((EPT,), jnp.int32),
        pltpu.VMEM((NP,), jnp.float32),
    ],
    compiler_params=_sc_cp,
)


_RB = 1024
_G = NP // _RB


def _tc_scale_body(hist, x, w, h1p, dis):
    ones32 = jnp.ones((NW, 1), jnp.float32)
    cnt = lax.dot_general(hist[...], ones32,
                          dimension_numbers=(((0,), (0,)), ((), ())),
                          preferred_element_type=jnp.float32)
    d = lax.rsqrt(cnt + 1.0)
    h = jnp.dot(x[...], w[...], preferred_element_type=jnp.float32)
    h1p[...] = h * d
    dis[...] = d


_tc_scale = pl.pallas_call(
    _tc_scale_body,
    grid=(_G,),
    in_specs=[
        pl.BlockSpec((NW, _RB), lambda i: (0, i)),
        pl.BlockSpec((_RB, D), lambda i: (i, 0)),
        pl.BlockSpec((D, D), lambda i: (0, 0)),
    ],
    out_specs=[
        pl.BlockSpec((_RB, D), lambda i: (i, 0)),
        pl.BlockSpec((_RB, 1), lambda i: (i, 0)),
    ],
    out_shape=[
        jax.ShapeDtypeStruct((N, D), jnp.float32),
        jax.ShapeDtypeStruct((N, 1), jnp.float32),
    ],
)


def _tc_mid_body(aggp, h1p, dis, b1, w2, h2p):
    s = aggp[0] + aggp[1] + h1p[...]
    out1 = s * dis[...] + b1[...]
    h2 = jnp.dot(out1, w2[...], preferred_element_type=jnp.float32)
    h2p[...] = h2 * dis[...]


_tc_mid = pl.pallas_call(
    _tc_mid_body,
    grid=(_G,),
    in_specs=[
        pl.BlockSpec((NC, _RB, D), lambda i: (0, i, 0)),
        pl.BlockSpec((_RB, D), lambda i: (i, 0)),
        pl.BlockSpec((_RB, 1), lambda i: (i, 0)),
        pl.BlockSpec((1, D), lambda i: (0, 0)),
        pl.BlockSpec((D, D), lambda i: (0, 0)),
    ],
    out_specs=pl.BlockSpec((_RB, D), lambda i: (i, 0)),
    out_shape=jax.ShapeDtypeStruct((N, D), jnp.float32),
)


def _tc_final_body(aggp, h2p, dis, b2, out):
    s = aggp[0] + aggp[1] + h2p[...]
    z = s * dis[...] + b2[...]
    m = jnp.max(z, axis=-1, keepdims=True)
    lse = jnp.log(jnp.sum(jnp.exp(z - m), axis=-1, keepdims=True)) + m
    out[...] = z - lse


_tc_final = pl.pallas_call(
    _tc_final_body,
    grid=(_G,),
    in_specs=[
        pl.BlockSpec((NC, _RB, D), lambda i: (0, i, 0)),
        pl.BlockSpec((_RB, D), lambda i: (i, 0)),
        pl.BlockSpec((_RB, 1), lambda i: (i, 0)),
        pl.BlockSpec((1, D), lambda i: (0, 0)),
    ],
    out_specs=pl.BlockSpec((_RB, D), lambda i: (i, 0)),
    out_shape=jax.ShapeDtypeStruct((N, D), jnp.float32),
)


def kernel(x, edge_index, W1, b1, W2, b2):
    ei = edge_index.astype(jnp.int32)
    row = ei[0]
    col = ei[1]
    npad = EPAD - E
    spread = jnp.arange(npad, dtype=jnp.int32)
    rowp = jnp.concatenate([row, spread % N])
    colp = jnp.concatenate([col, N + spread % (NP - N)])
    hist = _sc_hist(colp).reshape(NW, NP)
    h1p, dis = _tc_scale(hist, x, W1)
    agg1 = _sc_agg(h1p, rowp, colp)
    h2p = _tc_mid(agg1, h1p, dis, b1.reshape(1, D), W2)
    agg2 = _sc_agg(h2p, rowp, colp)
    return _tc_final(agg2, h2p, dis, b2.reshape(1, D))
